# baseline (speedup 1.0000x reference)
import jax
import jax.numpy as jnp
from jax.experimental import pallas as pl
from jax.experimental.pallas import tpu as pltpu

B = 1024
K = 65536
D = 128
C = 1000
CPAD = 1024
KBLK = 2048
NBLK = K // KBLK

_HI = jax.lax.Precision.HIGHEST


def _centroid_kernel(labels_ref, q_ref, sums_ref, counts_ref):
    i = pl.program_id(0)
    lab = labels_ref[0, 0, :]
    q = q_ref[...]
    cls = jax.lax.broadcasted_iota(jnp.int32, (CPAD, KBLK), 0)
    onehot = (cls == lab[None, :]).astype(jnp.float32)
    psum = jax.lax.dot_general(
        onehot, q, (((1,), (0,)), ((), ())),
        preferred_element_type=jnp.float32, precision=_HI)
    pcnt = jnp.sum(onehot, axis=1)[None, :]

    @pl.when(i == 0)
    def _():
        sums_ref[...] = psum
        counts_ref[...] = pcnt

    @pl.when(i > 0)
    def _():
        sums_ref[...] += psum
        counts_ref[...] += pcnt


def _label_kernel(sums_ref, counts_ref, bf_ref, p_ref, cnt_ref):
    sums = sums_ref[...]
    counts = counts_ref[0, :]
    cent = sums / jnp.maximum(counts, 1.0)[:, None]
    norm = jnp.sqrt(jnp.sum(cent * cent, axis=1, keepdims=True))
    centn = cent / jnp.maximum(norm, 1e-12)
    sim = jax.lax.dot_general(
        bf_ref[...], centn, (((1,), (1,)), ((), ())),
        preferred_element_type=jnp.float32, precision=_HI)
    col = jax.lax.broadcasted_iota(jnp.int32, (B, CPAD), 1)
    sim = jnp.where(col < C, sim, -3.0)
    p = jnp.argmax(sim, axis=1).astype(jnp.int32)
    p_ref[0, :] = p
    onehot = (p[:, None] == col).astype(jnp.float32)
    cnt_ref[0, :] = jnp.sum(onehot * counts[None, :], axis=1)


def _loss_kernel(bf_ref, q_ref, labels_ref, p_ref, cnt_ref, out_ref,
                 masked_acc, total_acc):
    i = pl.program_id(0)

    @pl.when(i == 0)
    def _():
        masked_acc[...] = jnp.zeros_like(masked_acc)
        total_acc[...] = jnp.zeros_like(total_acc)

    sim = jax.lax.dot_general(
        bf_ref[...], q_ref[...], (((1,), (1,)), ((), ())),
        preferred_element_type=jnp.float32, precision=_HI)
    mae = jnp.sqrt(2.0 - 2.0 * sim + 1e-6)
    lab = labels_ref[0, 0, :]
    p = p_ref[0, :]
    mask = p[:, None] == lab[None, :]
    masked_acc[0, :] += jnp.sum(jnp.where(mask, mae, 0.0), axis=1)
    total_acc[0, :] += jnp.sum(mae, axis=1)

    @pl.when(i == NBLK - 1)
    def _():
        m = masked_acc[0, :]
        t = total_acc[0, :]
        cnt = cnt_ref[0, :]
        minent = jnp.mean(m / (cnt + 1e-6))
        inter = jnp.mean((t - m) / ((jnp.float32(K) - cnt) + 1e-6))
        out_ref[...] = jnp.broadcast_to(minent + 2.0 - inter, (1, 1))


def kernel(batch_feature, queue_emb_copy, info_label):
    labels3 = info_label.astype(jnp.int32).reshape(NBLK, 1, KBLK)

    sums, counts = pl.pallas_call(
        _centroid_kernel,
        grid=(NBLK,),
        in_specs=[
            pl.BlockSpec((1, 1, KBLK), lambda i: (i, 0, 0)),
            pl.BlockSpec((KBLK, D), lambda i: (i, 0)),
        ],
        out_specs=[
            pl.BlockSpec((CPAD, D), lambda i: (0, 0)),
            pl.BlockSpec((1, CPAD), lambda i: (0, 0)),
        ],
        out_shape=[
            jax.ShapeDtypeStruct((CPAD, D), jnp.float32),
            jax.ShapeDtypeStruct((1, CPAD), jnp.float32),
        ],
    )(labels3, queue_emb_copy)

    p, cnt = pl.pallas_call(
        _label_kernel,
        in_specs=[
            pl.BlockSpec((CPAD, D), lambda: (0, 0)),
            pl.BlockSpec((1, CPAD), lambda: (0, 0)),
            pl.BlockSpec((B, D), lambda: (0, 0)),
        ],
        out_specs=[
            pl.BlockSpec((1, B), lambda: (0, 0)),
            pl.BlockSpec((1, B), lambda: (0, 0)),
        ],
        out_shape=[
            jax.ShapeDtypeStruct((1, B), jnp.int32),
            jax.ShapeDtypeStruct((1, B), jnp.float32),
        ],
    )(sums, counts, batch_feature)

    out = pl.pallas_call(
        _loss_kernel,
        grid=(NBLK,),
        in_specs=[
            pl.BlockSpec((B, D), lambda i: (0, 0)),
            pl.BlockSpec((KBLK, D), lambda i: (i, 0)),
            pl.BlockSpec((1, 1, KBLK), lambda i: (i, 0, 0)),
            pl.BlockSpec((1, B), lambda i: (0, 0)),
            pl.BlockSpec((1, B), lambda i: (0, 0)),
        ],
        out_specs=pl.BlockSpec((1, 1), lambda i: (0, 0)),
        out_shape=jax.ShapeDtypeStruct((1, 1), jnp.float32),
        scratch_shapes=[
            pltpu.VMEM((1, B), jnp.float32),
            pltpu.VMEM((1, B), jnp.float32),
        ],
    )(batch_feature, queue_emb_copy, labels3, p, cnt)

    return out[0, 0]


# speedup vs baseline: 1.9499x; 1.9499x over previous
import jax
import jax.numpy as jnp
from jax.experimental import pallas as pl
from jax.experimental.pallas import tpu as pltpu

B = 1024
K = 65536
D = 128
C = 1000
CPAD = 1024
KBLK = 2048
NBLK = K // KBLK

_HI = jax.lax.Precision.HIGHEST


def _centroid_kernel(labels_ref, q_ref, sums_ref, counts_ref):
    i = pl.program_id(0)
    lab = labels_ref[0, 0, :]
    q = q_ref[...]
    cls = jax.lax.broadcasted_iota(jnp.int32, (CPAD, KBLK), 0)
    hit = cls == lab[None, :]
    onehot = hit.astype(jnp.bfloat16)
    psum = jax.lax.dot_general(
        onehot, q, (((1,), (0,)), ((), ())),
        preferred_element_type=jnp.float32)
    pcnt = jnp.sum(hit.astype(jnp.float32), axis=1)[None, :]

    @pl.when(i == 0)
    def _():
        sums_ref[...] = psum
        counts_ref[...] = pcnt

    @pl.when(i > 0)
    def _():
        sums_ref[...] += psum
        counts_ref[...] += pcnt


def _label_kernel(sums_ref, counts_ref, bf_ref, p_ref, cnt_ref):
    sums = sums_ref[...]
    counts = counts_ref[0, :]
    cent = sums / jnp.maximum(counts, 1.0)[:, None]
    norm = jnp.sqrt(jnp.sum(cent * cent, axis=1, keepdims=True))
    centn = cent / jnp.maximum(norm, 1e-12)
    sim = jax.lax.dot_general(
        bf_ref[...], centn, (((1,), (1,)), ((), ())),
        preferred_element_type=jnp.float32, precision=_HI)
    col = jax.lax.broadcasted_iota(jnp.int32, (B, CPAD), 1)
    sim = jnp.where(col < C, sim, -3.0)
    p = jnp.argmax(sim, axis=1).astype(jnp.int32)
    p_ref[0, :] = p
    onehot = (p[:, None] == col).astype(jnp.float32)
    cnt_ref[0, :] = jnp.sum(onehot * counts[None, :], axis=1)


def _loss_kernel(bf_ref, q_ref, labels_ref, p_ref, cnt_ref, out_ref,
                 masked_acc, total_acc):
    i = pl.program_id(0)

    @pl.when(i == 0)
    def _():
        masked_acc[...] = jnp.zeros_like(masked_acc)
        total_acc[...] = jnp.zeros_like(total_acc)

    sim = jax.lax.dot_general(
        bf_ref[...], q_ref[...], (((1,), (1,)), ((), ())),
        preferred_element_type=jnp.float32)
    mae = jnp.sqrt(2.0 - 2.0 * sim + 1e-6)
    lab = labels_ref[0, 0, :]
    p = p_ref[0, :]
    mask = p[:, None] == lab[None, :]
    masked_acc[0, :] += jnp.sum(jnp.where(mask, mae, 0.0), axis=1)
    total_acc[0, :] += jnp.sum(mae, axis=1)

    @pl.when(i == NBLK - 1)
    def _():
        m = masked_acc[0, :]
        t = total_acc[0, :]
        cnt = cnt_ref[0, :]
        minent = jnp.mean(m / (cnt + 1e-6))
        inter = jnp.mean((t - m) / ((jnp.float32(K) - cnt) + 1e-6))
        out_ref[...] = jnp.broadcast_to(minent + 2.0 - inter, (1, 1))


def kernel(batch_feature, queue_emb_copy, info_label):
    labels3 = info_label.astype(jnp.int32).reshape(NBLK, 1, KBLK)
    q16 = queue_emb_copy.astype(jnp.bfloat16)
    bf16 = batch_feature.astype(jnp.bfloat16)

    sums, counts = pl.pallas_call(
        _centroid_kernel,
        grid=(NBLK,),
        in_specs=[
            pl.BlockSpec((1, 1, KBLK), lambda i: (i, 0, 0)),
            pl.BlockSpec((KBLK, D), lambda i: (i, 0)),
        ],
        out_specs=[
            pl.BlockSpec((CPAD, D), lambda i: (0, 0)),
            pl.BlockSpec((1, CPAD), lambda i: (0, 0)),
        ],
        out_shape=[
            jax.ShapeDtypeStruct((CPAD, D), jnp.float32),
            jax.ShapeDtypeStruct((1, CPAD), jnp.float32),
        ],
    )(labels3, q16)

    p, cnt = pl.pallas_call(
        _label_kernel,
        in_specs=[
            pl.BlockSpec((CPAD, D), lambda: (0, 0)),
            pl.BlockSpec((1, CPAD), lambda: (0, 0)),
            pl.BlockSpec((B, D), lambda: (0, 0)),
        ],
        out_specs=[
            pl.BlockSpec((1, B), lambda: (0, 0)),
            pl.BlockSpec((1, B), lambda: (0, 0)),
        ],
        out_shape=[
            jax.ShapeDtypeStruct((1, B), jnp.int32),
            jax.ShapeDtypeStruct((1, B), jnp.float32),
        ],
    )(sums, counts, batch_feature)

    out = pl.pallas_call(
        _loss_kernel,
        grid=(NBLK,),
        in_specs=[
            pl.BlockSpec((B, D), lambda i: (0, 0)),
            pl.BlockSpec((KBLK, D), lambda i: (i, 0)),
            pl.BlockSpec((1, 1, KBLK), lambda i: (i, 0, 0)),
            pl.BlockSpec((1, B), lambda i: (0, 0)),
            pl.BlockSpec((1, B), lambda i: (0, 0)),
        ],
        out_specs=pl.BlockSpec((1, 1), lambda i: (0, 0)),
        out_shape=jax.ShapeDtypeStruct((1, 1), jnp.float32),
        scratch_shapes=[
            pltpu.VMEM((1, B), jnp.float32),
            pltpu.VMEM((1, B), jnp.float32),
        ],
    )(bf16, q16, labels3, p, cnt)

    return out[0, 0]


# speedup vs baseline: 3.5352x; 1.8130x over previous
import functools

import jax
import jax.numpy as jnp
from jax.experimental import pallas as pl
from jax.experimental.pallas import tpu as pltpu

B = 1024
K = 65536
D = 128
C = 1000
CPAD = 1024
KBLK = 2048
NBLK = K // KBLK
NLG = KBLK // 128

_HI = jax.lax.Precision.HIGHEST


def _lanegroup_sum(x, width=128):
    parts = [x[:, g * width:(g + 1) * width] for g in range(x.shape[1] // width)]
    while len(parts) > 1:
        nxt = [parts[i] + parts[i + 1] for i in range(0, len(parts) - 1, 2)]
        if len(parts) % 2:
            nxt.append(parts[-1])
        parts = nxt
    return parts[0]


def _centroid_kernel(labels_ref, q_ref, sums_ref, cacc_ref):
    i = pl.program_id(0)
    lab = labels_ref[0, 0, :]
    cls = jax.lax.broadcasted_iota(jnp.int32, (CPAD, KBLK), 0)
    hit = cls == lab[None, :]
    onehot = hit.astype(jnp.bfloat16)
    psum = jax.lax.dot_general(
        onehot, q_ref[...], (((1,), (0,)), ((), ())),
        preferred_element_type=jnp.float32)
    pcnt = _lanegroup_sum(onehot).astype(jnp.float32)

    @pl.when(i == 0)
    def _():
        sums_ref[...] = psum
        cacc_ref[...] = pcnt

    @pl.when(i > 0)
    def _():
        sums_ref[...] += psum
        cacc_ref[...] += pcnt


def _label_kernel(sums_ref, cacc_ref, bf_ref, p_ref, cnt_ref):
    sums = sums_ref[...]
    s2 = jnp.sum(sums * sums, axis=1, keepdims=True)
    centn = sums / jnp.maximum(jnp.sqrt(s2), 1e-12)
    sim = jax.lax.dot_general(
        bf_ref[...], centn, (((1,), (1,)), ((), ())),
        preferred_element_type=jnp.float32, precision=_HI)
    col = jax.lax.broadcasted_iota(jnp.int32, (B, CPAD), 1)
    sim = jnp.where(col < C, sim, -3.0)
    p = jnp.argmax(sim, axis=1).astype(jnp.int32)
    p_ref[...] = p[:, None]
    onehot_p = (p[:, None] == col).astype(jnp.bfloat16)
    cntm = jax.lax.dot_general(
        onehot_p, cacc_ref[...].astype(jnp.bfloat16), (((1,), (0,)), ((), ())),
        preferred_element_type=jnp.float32)
    cnt_ref[...] = jnp.sum(cntm, axis=1, keepdims=True)


def _loss_kernel(bfm2_ref, q_ref, labels_ref, p_ref, cnt_ref, out_ref,
                 macc, tacc):
    i = pl.program_id(0)

    @pl.when(i == 0)
    def _():
        macc[...] = jnp.zeros_like(macc)
        tacc[...] = jnp.zeros_like(tacc)

    simn2 = jax.lax.dot_general(
        bfm2_ref[...], q_ref[...], (((1,), (1,)), ((), ())),
        preferred_element_type=jnp.float32)
    t = simn2 + jnp.float32(2.0 + 1e-6)
    mae = t * jax.lax.rsqrt(t)
    lab = labels_ref[0, 0, :]
    p = p_ref[...]
    mask = p == lab[None, :]
    macc[...] += _lanegroup_sum(jnp.where(mask, mae, 0.0))
    tacc[...] += _lanegroup_sum(mae)

    @pl.when(i == NBLK - 1)
    def _():
        m = jnp.sum(macc[...], axis=1)
        t_ = jnp.sum(tacc[...], axis=1)
        cnt = cnt_ref[:, 0]
        minent = jnp.mean(m / (cnt + 1e-6))
        inter = jnp.mean((t_ - m) / ((jnp.float32(K) - cnt) + 1e-6))
        out_ref[...] = jnp.broadcast_to(minent + 2.0 - inter, (1, 1))


def kernel(batch_feature, queue_emb_copy, info_label):
    labels3 = info_label.astype(jnp.int32).reshape(NBLK, 1, KBLK)
    q16 = queue_emb_copy.astype(jnp.bfloat16)
    bfm2 = (batch_feature * -2.0).astype(jnp.bfloat16)

    sums, cacc = pl.pallas_call(
        _centroid_kernel,
        grid=(NBLK,),
        in_specs=[
            pl.BlockSpec((1, 1, KBLK), lambda i: (i, 0, 0)),
            pl.BlockSpec((KBLK, D), lambda i: (i, 0)),
        ],
        out_specs=[
            pl.BlockSpec((CPAD, D), lambda i: (0, 0)),
            pl.BlockSpec((CPAD, 128), lambda i: (0, 0)),
        ],
        out_shape=[
            jax.ShapeDtypeStruct((CPAD, D), jnp.float32),
            jax.ShapeDtypeStruct((CPAD, 128), jnp.float32),
        ],
    )(labels3, q16)

    p, cnt = pl.pallas_call(
        _label_kernel,
        in_specs=[
            pl.BlockSpec((CPAD, D), lambda: (0, 0)),
            pl.BlockSpec((CPAD, 128), lambda: (0, 0)),
            pl.BlockSpec((B, D), lambda: (0, 0)),
        ],
        out_specs=[
            pl.BlockSpec((B, 1), lambda: (0, 0)),
            pl.BlockSpec((B, 1), lambda: (0, 0)),
        ],
        out_shape=[
            jax.ShapeDtypeStruct((B, 1), jnp.int32),
            jax.ShapeDtypeStruct((B, 1), jnp.float32),
        ],
    )(sums, cacc, batch_feature)

    out = pl.pallas_call(
        _loss_kernel,
        grid=(NBLK,),
        in_specs=[
            pl.BlockSpec((B, D), lambda i: (0, 0)),
            pl.BlockSpec((KBLK, D), lambda i: (i, 0)),
            pl.BlockSpec((1, 1, KBLK), lambda i: (i, 0, 0)),
            pl.BlockSpec((B, 1), lambda i: (0, 0)),
            pl.BlockSpec((B, 1), lambda i: (0, 0)),
        ],
        out_specs=pl.BlockSpec((1, 1), lambda i: (0, 0)),
        out_shape=jax.ShapeDtypeStruct((1, 1), jnp.float32),
        scratch_shapes=[
            pltpu.VMEM((B, 128), jnp.float32),
            pltpu.VMEM((B, 128), jnp.float32),
        ],
    )(bfm2, q16, labels3, p, cnt)

    return out[0, 0]


# speedup vs baseline: 3.8410x; 1.0865x over previous
import jax
import jax.numpy as jnp
from jax.experimental import pallas as pl
from jax.experimental.pallas import tpu as pltpu

B = 1024
K = 65536
D = 128
C = 1000
CPAD = 1024
KBLK = 2048
NBLK = K // KBLK

_HI = jax.lax.Precision.HIGHEST


def _lanegroup_sum(x, width=128):
    parts = [x[:, g * width:(g + 1) * width] for g in range(x.shape[1] // width)]
    while len(parts) > 1:
        nxt = [parts[i] + parts[i + 1] for i in range(0, len(parts) - 1, 2)]
        if len(parts) % 2:
            nxt.append(parts[-1])
        parts = nxt
    return parts[0]


def _centroid_kernel(labels_ref, q_ref, sums_ref, cacc_ref):
    i = pl.program_id(0)
    lab = labels_ref[0, 0, :]
    cls = jax.lax.broadcasted_iota(jnp.int16, (CPAD, KBLK), 0)
    hit = cls == lab[None, :]
    onehot = jnp.where(hit, jnp.bfloat16(1.0), jnp.bfloat16(0.0))
    psum = jax.lax.dot_general(
        onehot, q_ref[...], (((1,), (0,)), ((), ())),
        preferred_element_type=jnp.float32)
    pcnt = _lanegroup_sum(onehot).astype(jnp.float32)

    @pl.when(i == 0)
    def _():
        sums_ref[...] = psum
        cacc_ref[...] = pcnt

    @pl.when(i > 0)
    def _():
        sums_ref[...] += psum
        cacc_ref[...] += pcnt


def _label_kernel(sums_ref, cacc_ref, bf_ref, p_ref, cnt_ref):
    sums = sums_ref[...]
    s2 = jnp.sum(sums * sums, axis=1, keepdims=True)
    centn = sums / jnp.maximum(jnp.sqrt(s2), 1e-12)
    sim = jax.lax.dot_general(
        bf_ref[...], centn, (((1,), (1,)), ((), ())),
        preferred_element_type=jnp.float32, precision=_HI)
    col = jax.lax.broadcasted_iota(jnp.int32, (B, CPAD), 1)
    sim = jnp.where(col < C, sim, -3.0)
    p = jnp.argmax(sim, axis=1).astype(jnp.int32)
    p_ref[...] = p[:, None]
    onehot_p = (p[:, None] == col).astype(jnp.bfloat16)
    cntm = jax.lax.dot_general(
        onehot_p, cacc_ref[...].astype(jnp.bfloat16), (((1,), (0,)), ((), ())),
        preferred_element_type=jnp.float32)
    cnt_ref[...] = jnp.sum(cntm, axis=1, keepdims=True)


def _loss_kernel(bfm2_ref, q_ref, labels_ref, p_ref, cnt_ref, out_ref,
                 macc, tacc):
    i = pl.program_id(0)

    @pl.when(i == 0)
    def _():
        macc[...] = jnp.zeros_like(macc)
        tacc[...] = jnp.zeros_like(tacc)

    simn2 = jax.lax.dot_general(
        bfm2_ref[...], q_ref[...], (((1,), (1,)), ((), ())),
        preferred_element_type=jnp.float32)
    t = simn2 + jnp.float32(2.0 + 1e-6)
    mae = (t * jax.lax.rsqrt(t)).astype(jnp.bfloat16)
    lab = labels_ref[0, 0, :]
    p16 = p_ref[...].astype(jnp.int16)
    mask = p16 == lab[None, :]
    macc[...] += _lanegroup_sum(
        jnp.where(mask, mae, jnp.bfloat16(0.0))).astype(jnp.float32)
    tacc[...] += _lanegroup_sum(mae).astype(jnp.float32)

    @pl.when(i == NBLK - 1)
    def _():
        m = jnp.sum(macc[...], axis=1)
        t_ = jnp.sum(tacc[...], axis=1)
        cnt = cnt_ref[:, 0]
        minent = jnp.mean(m / (cnt + 1e-6))
        inter = jnp.mean((t_ - m) / ((jnp.float32(K) - cnt) + 1e-6))
        out_ref[...] = jnp.broadcast_to(minent + 2.0 - inter, (1, 1))


def kernel(batch_feature, queue_emb_copy, info_label):
    labels3 = info_label.astype(jnp.int16).reshape(NBLK, 1, KBLK)
    q16 = queue_emb_copy.astype(jnp.bfloat16)
    bfm2 = (batch_feature * -2.0).astype(jnp.bfloat16)

    sums, cacc = pl.pallas_call(
        _centroid_kernel,
        grid=(NBLK,),
        in_specs=[
            pl.BlockSpec((1, 1, KBLK), lambda i: (i, 0, 0)),
            pl.BlockSpec((KBLK, D), lambda i: (i, 0)),
        ],
        out_specs=[
            pl.BlockSpec((CPAD, D), lambda i: (0, 0)),
            pl.BlockSpec((CPAD, 128), lambda i: (0, 0)),
        ],
        out_shape=[
            jax.ShapeDtypeStruct((CPAD, D), jnp.float32),
            jax.ShapeDtypeStruct((CPAD, 128), jnp.float32),
        ],
    )(labels3, q16)

    p, cnt = pl.pallas_call(
        _label_kernel,
        in_specs=[
            pl.BlockSpec((CPAD, D), lambda: (0, 0)),
            pl.BlockSpec((CPAD, 128), lambda: (0, 0)),
            pl.BlockSpec((B, D), lambda: (0, 0)),
        ],
        out_specs=[
            pl.BlockSpec((B, 1), lambda: (0, 0)),
            pl.BlockSpec((B, 1), lambda: (0, 0)),
        ],
        out_shape=[
            jax.ShapeDtypeStruct((B, 1), jnp.int32),
            jax.ShapeDtypeStruct((B, 1), jnp.float32),
        ],
    )(sums, cacc, batch_feature)

    out = pl.pallas_call(
        _loss_kernel,
        grid=(NBLK,),
        in_specs=[
            pl.BlockSpec((B, D), lambda i: (0, 0)),
            pl.BlockSpec((KBLK, D), lambda i: (i, 0)),
            pl.BlockSpec((1, 1, KBLK), lambda i: (i, 0, 0)),
            pl.BlockSpec((B, 1), lambda i: (0, 0)),
            pl.BlockSpec((B, 1), lambda i: (0, 0)),
        ],
        out_specs=pl.BlockSpec((1, 1), lambda i: (0, 0)),
        out_shape=jax.ShapeDtypeStruct((1, 1), jnp.float32),
        scratch_shapes=[
            pltpu.VMEM((B, 128), jnp.float32),
            pltpu.VMEM((B, 128), jnp.float32),
        ],
    )(bfm2, q16, labels3, p, cnt)

    return out[0, 0]


# speedup vs baseline: 3.9580x; 1.0305x over previous
import jax
import jax.numpy as jnp
from jax.experimental import pallas as pl
from jax.experimental.pallas import tpu as pltpu

B = 1024
K = 65536
D = 128
C = 1000
CPAD = 1024
KBLK = 4096
NBLK = K // KBLK

_HI = jax.lax.Precision.HIGHEST


def _lanegroup_sum(x, width=128):
    parts = [x[:, g * width:(g + 1) * width] for g in range(x.shape[1] // width)]
    while len(parts) > 1:
        nxt = [parts[i] + parts[i + 1] for i in range(0, len(parts) - 1, 2)]
        if len(parts) % 2:
            nxt.append(parts[-1])
        parts = nxt
    return parts[0]


def _centroid_kernel(labels_ref, q_ref, sums_ref, cacc_ref):
    i = pl.program_id(0)
    lab = labels_ref[0, 0, :]
    cls = jax.lax.broadcasted_iota(jnp.int16, (CPAD, KBLK), 0)
    hit = cls == lab[None, :]
    onehot = jnp.where(hit, jnp.bfloat16(1.0), jnp.bfloat16(0.0))
    psum = jax.lax.dot_general(
        onehot, q_ref[...], (((1,), (0,)), ((), ())),
        preferred_element_type=jnp.float32)
    pcnt = _lanegroup_sum(onehot).astype(jnp.float32)

    @pl.when(i == 0)
    def _():
        sums_ref[...] = psum
        cacc_ref[...] = pcnt

    @pl.when(i > 0)
    def _():
        sums_ref[...] += psum
        cacc_ref[...] += pcnt


def _label_kernel(sums_ref, cacc_ref, bf_ref, p_ref, cnt_ref):
    sums = sums_ref[...]
    s2 = jnp.sum(sums * sums, axis=1, keepdims=True)
    centn = sums / jnp.maximum(jnp.sqrt(s2), 1e-12)
    sim = jax.lax.dot_general(
        bf_ref[...], centn, (((1,), (1,)), ((), ())),
        preferred_element_type=jnp.float32, precision=_HI)
    col = jax.lax.broadcasted_iota(jnp.int32, (B, CPAD), 1)
    sim = jnp.where(col < C, sim, -3.0)
    p = jnp.argmax(sim, axis=1).astype(jnp.int32)
    p_ref[...] = p[:, None]
    onehot_p = (p[:, None] == col).astype(jnp.bfloat16)
    cntm = jax.lax.dot_general(
        onehot_p, cacc_ref[...].astype(jnp.bfloat16), (((1,), (0,)), ((), ())),
        preferred_element_type=jnp.float32)
    cnt_ref[...] = jnp.sum(cntm, axis=1, keepdims=True)


def _loss_kernel(bfm2_ref, q_ref, labels_ref, p_ref, cnt_ref, out_ref,
                 macc, tacc):
    i = pl.program_id(0)

    @pl.when(i == 0)
    def _():
        macc[...] = jnp.zeros_like(macc)
        tacc[...] = jnp.zeros_like(tacc)

    simn2 = jax.lax.dot_general(
        bfm2_ref[...], q_ref[...], (((1,), (1,)), ((), ())),
        preferred_element_type=jnp.float32)
    t = simn2 + jnp.float32(2.0 + 1e-6)
    mae = (t * jax.lax.rsqrt(t)).astype(jnp.bfloat16)
    lab = labels_ref[0, 0, :]
    p16 = p_ref[...].astype(jnp.int16)
    mask = p16 == lab[None, :]
    macc[...] += _lanegroup_sum(
        jnp.where(mask, mae, jnp.bfloat16(0.0))).astype(jnp.float32)
    tacc[...] += _lanegroup_sum(mae).astype(jnp.float32)

    @pl.when(i == NBLK - 1)
    def _():
        m = jnp.sum(macc[...], axis=1)
        t_ = jnp.sum(tacc[...], axis=1)
        cnt = cnt_ref[:, 0]
        minent = jnp.mean(m / (cnt + 1e-6))
        inter = jnp.mean((t_ - m) / ((jnp.float32(K) - cnt) + 1e-6))
        out_ref[...] = jnp.broadcast_to(minent + 2.0 - inter, (1, 1))


def kernel(batch_feature, queue_emb_copy, info_label):
    labels3 = info_label.astype(jnp.int16).reshape(NBLK, 1, KBLK)
    q16 = queue_emb_copy.astype(jnp.bfloat16)
    bfm2 = (batch_feature * -2.0).astype(jnp.bfloat16)

    sums, cacc = pl.pallas_call(
        _centroid_kernel,
        grid=(NBLK,),
        in_specs=[
            pl.BlockSpec((1, 1, KBLK), lambda i: (i, 0, 0)),
            pl.BlockSpec((KBLK, D), lambda i: (i, 0)),
        ],
        out_specs=[
            pl.BlockSpec((CPAD, D), lambda i: (0, 0)),
            pl.BlockSpec((CPAD, 128), lambda i: (0, 0)),
        ],
        out_shape=[
            jax.ShapeDtypeStruct((CPAD, D), jnp.float32),
            jax.ShapeDtypeStruct((CPAD, 128), jnp.float32),
        ],
    )(labels3, q16)

    p, cnt = pl.pallas_call(
        _label_kernel,
        in_specs=[
            pl.BlockSpec((CPAD, D), lambda: (0, 0)),
            pl.BlockSpec((CPAD, 128), lambda: (0, 0)),
            pl.BlockSpec((B, D), lambda: (0, 0)),
        ],
        out_specs=[
            pl.BlockSpec((B, 1), lambda: (0, 0)),
            pl.BlockSpec((B, 1), lambda: (0, 0)),
        ],
        out_shape=[
            jax.ShapeDtypeStruct((B, 1), jnp.int32),
            jax.ShapeDtypeStruct((B, 1), jnp.float32),
        ],
    )(sums, cacc, batch_feature)

    out = pl.pallas_call(
        _loss_kernel,
        grid=(NBLK,),
        in_specs=[
            pl.BlockSpec((B, D), lambda i: (0, 0)),
            pl.BlockSpec((KBLK, D), lambda i: (i, 0)),
            pl.BlockSpec((1, 1, KBLK), lambda i: (i, 0, 0)),
            pl.BlockSpec((B, 1), lambda i: (0, 0)),
            pl.BlockSpec((B, 1), lambda i: (0, 0)),
        ],
        out_specs=pl.BlockSpec((1, 1), lambda i: (0, 0)),
        out_shape=jax.ShapeDtypeStruct((1, 1), jnp.float32),
        scratch_shapes=[
            pltpu.VMEM((B, 128), jnp.float32),
            pltpu.VMEM((B, 128), jnp.float32),
        ],
    )(bfm2, q16, labels3, p, cnt)

    return out[0, 0]


# speedup vs baseline: 3.9917x; 1.0085x over previous
import jax
import jax.numpy as jnp
from jax.experimental import pallas as pl
from jax.experimental.pallas import tpu as pltpu

B = 1024
K = 65536
D = 128
C = 1000
CPAD = 1024
KBLK = 8192
NBLK = K // KBLK

_HI = jax.lax.Precision.HIGHEST


def _lanegroup_sum(x, width=128):
    parts = [x[:, g * width:(g + 1) * width] for g in range(x.shape[1] // width)]
    while len(parts) > 1:
        nxt = [parts[i] + parts[i + 1] for i in range(0, len(parts) - 1, 2)]
        if len(parts) % 2:
            nxt.append(parts[-1])
        parts = nxt
    return parts[0]


def _centroid_kernel(labels_ref, q_ref, sums_ref, cacc_ref):
    i = pl.program_id(0)
    lab = labels_ref[0, 0, :]
    cls = jax.lax.broadcasted_iota(jnp.int16, (CPAD, KBLK), 0)
    hit = cls == lab[None, :]
    onehot = jnp.where(hit, jnp.bfloat16(1.0), jnp.bfloat16(0.0))
    psum = jax.lax.dot_general(
        onehot, q_ref[...], (((1,), (0,)), ((), ())),
        preferred_element_type=jnp.float32)
    pcnt = _lanegroup_sum(onehot).astype(jnp.float32)

    @pl.when(i == 0)
    def _():
        sums_ref[...] = psum
        cacc_ref[...] = pcnt

    @pl.when(i > 0)
    def _():
        sums_ref[...] += psum
        cacc_ref[...] += pcnt


def _label_kernel(sums_ref, cacc_ref, bf_ref, p_ref, cnt_ref):
    sums = sums_ref[...]
    s2 = jnp.sum(sums * sums, axis=1, keepdims=True)
    centn = sums / jnp.maximum(jnp.sqrt(s2), 1e-12)
    sim = jax.lax.dot_general(
        bf_ref[...], centn, (((1,), (1,)), ((), ())),
        preferred_element_type=jnp.float32, precision=_HI)
    col = jax.lax.broadcasted_iota(jnp.int32, (B, CPAD), 1)
    sim = jnp.where(col < C, sim, -3.0)
    p = jnp.argmax(sim, axis=1).astype(jnp.int32)
    p_ref[...] = p[:, None]
    onehot_p = (p[:, None] == col).astype(jnp.bfloat16)
    cntm = jax.lax.dot_general(
        onehot_p, cacc_ref[...].astype(jnp.bfloat16), (((1,), (0,)), ((), ())),
        preferred_element_type=jnp.float32)
    cnt_ref[...] = jnp.sum(cntm, axis=1, keepdims=True)


def _loss_kernel(bfm2_ref, q_ref, labels_ref, p_ref, cnt_ref, out_ref,
                 macc, tacc):
    i = pl.program_id(0)

    @pl.when(i == 0)
    def _():
        macc[...] = jnp.zeros_like(macc)
        tacc[...] = jnp.zeros_like(tacc)

    simn2 = jax.lax.dot_general(
        bfm2_ref[...], q_ref[...], (((1,), (1,)), ((), ())),
        preferred_element_type=jnp.float32)
    t = simn2 + jnp.float32(2.0 + 1e-6)
    mae = (t * jax.lax.rsqrt(t)).astype(jnp.bfloat16)
    lab = labels_ref[0, 0, :]
    p16 = p_ref[...].astype(jnp.int16)
    mask = p16 == lab[None, :]
    macc[...] += _lanegroup_sum(
        jnp.where(mask, mae, jnp.bfloat16(0.0))).astype(jnp.float32)
    tacc[...] += _lanegroup_sum(mae).astype(jnp.float32)

    @pl.when(i == NBLK - 1)
    def _():
        m = jnp.sum(macc[...], axis=1)
        t_ = jnp.sum(tacc[...], axis=1)
        cnt = cnt_ref[:, 0]
        minent = jnp.mean(m / (cnt + 1e-6))
        inter = jnp.mean((t_ - m) / ((jnp.float32(K) - cnt) + 1e-6))
        out_ref[...] = jnp.broadcast_to(minent + 2.0 - inter, (1, 1))


def kernel(batch_feature, queue_emb_copy, info_label):
    labels3 = info_label.astype(jnp.int16).reshape(NBLK, 1, KBLK)
    q16 = queue_emb_copy.astype(jnp.bfloat16)
    bfm2 = (batch_feature * -2.0).astype(jnp.bfloat16)

    sums, cacc = pl.pallas_call(
        _centroid_kernel,
        grid=(NBLK,),
        in_specs=[
            pl.BlockSpec((1, 1, KBLK), lambda i: (i, 0, 0)),
            pl.BlockSpec((KBLK, D), lambda i: (i, 0)),
        ],
        out_specs=[
            pl.BlockSpec((CPAD, D), lambda i: (0, 0)),
            pl.BlockSpec((CPAD, 128), lambda i: (0, 0)),
        ],
        out_shape=[
            jax.ShapeDtypeStruct((CPAD, D), jnp.float32),
            jax.ShapeDtypeStruct((CPAD, 128), jnp.float32),
        ],
    )(labels3, q16)

    p, cnt = pl.pallas_call(
        _label_kernel,
        in_specs=[
            pl.BlockSpec((CPAD, D), lambda: (0, 0)),
            pl.BlockSpec((CPAD, 128), lambda: (0, 0)),
            pl.BlockSpec((B, D), lambda: (0, 0)),
        ],
        out_specs=[
            pl.BlockSpec((B, 1), lambda: (0, 0)),
            pl.BlockSpec((B, 1), lambda: (0, 0)),
        ],
        out_shape=[
            jax.ShapeDtypeStruct((B, 1), jnp.int32),
            jax.ShapeDtypeStruct((B, 1), jnp.float32),
        ],
    )(sums, cacc, batch_feature)

    out = pl.pallas_call(
        _loss_kernel,
        grid=(NBLK,),
        in_specs=[
            pl.BlockSpec((B, D), lambda i: (0, 0)),
            pl.BlockSpec((KBLK, D), lambda i: (i, 0)),
            pl.BlockSpec((1, 1, KBLK), lambda i: (i, 0, 0)),
            pl.BlockSpec((B, 1), lambda i: (0, 0)),
            pl.BlockSpec((B, 1), lambda i: (0, 0)),
        ],
        out_specs=pl.BlockSpec((1, 1), lambda i: (0, 0)),
        out_shape=jax.ShapeDtypeStruct((1, 1), jnp.float32),
        scratch_shapes=[
            pltpu.VMEM((B, 128), jnp.float32),
            pltpu.VMEM((B, 128), jnp.float32),
        ],
    )(bfm2, q16, labels3, p, cnt)

    return out[0, 0]


# speedup vs baseline: 4.1070x; 1.0289x over previous
import jax
import jax.numpy as jnp
from jax.experimental import pallas as pl
from jax.experimental.pallas import tpu as pltpu

B = 1024
K = 65536
D = 128
C = 1000
CPAD = 1024
KBLK = 4096
NBLK = K // KBLK


def _lanegroup_sum(x, width=128):
    parts = [x[:, g * width:(g + 1) * width] for g in range(x.shape[1] // width)]
    while len(parts) > 1:
        nxt = [parts[i] + parts[i + 1] for i in range(0, len(parts) - 1, 2)]
        if len(parts) % 2:
            nxt.append(parts[-1])
        parts = nxt
    return parts[0]


def _fused_kernel(labels_ref, q_ref, bfm2_ref, out_ref,
                  sums_s, cacc_s, p_s, cnt_s, macc, tacc):
    i = pl.program_id(0)

    @pl.when(i < NBLK)
    def _():
        lab = labels_ref[0, 0, :]
        cls = jax.lax.broadcasted_iota(jnp.int16, (CPAD, KBLK), 0)
        hit = cls == lab[None, :]
        onehot = jnp.where(hit, jnp.bfloat16(1.0), jnp.bfloat16(0.0))
        psum = jax.lax.dot_general(
            onehot, q_ref[...], (((1,), (0,)), ((), ())),
            preferred_element_type=jnp.float32)
        pcnt = _lanegroup_sum(onehot).astype(jnp.float32)

        @pl.when(i == 0)
        def _():
            sums_s[...] = psum
            cacc_s[...] = pcnt

        @pl.when(i > 0)
        def _():
            sums_s[...] += psum
            cacc_s[...] += pcnt

    @pl.when(i == NBLK)
    def _():
        sums = sums_s[...]
        s2 = jnp.sum(sums * sums, axis=1, keepdims=True)
        centn = (sums * jax.lax.rsqrt(jnp.maximum(s2, 1e-24))).astype(
            jnp.bfloat16)
        bf = bfm2_ref[...] * jnp.bfloat16(-0.5)
        sim = jax.lax.dot_general(
            bf, centn, (((1,), (1,)), ((), ())),
            preferred_element_type=jnp.float32)
        col = jax.lax.broadcasted_iota(jnp.int32, (B, CPAD), 1)
        sim = jnp.where(col < C, sim, -3.0)
        p = jnp.argmax(sim, axis=1).astype(jnp.int32)
        p_s[...] = p[:, None]
        onehot_p = (p[:, None] == col).astype(jnp.bfloat16)
        cntm = jax.lax.dot_general(
            onehot_p, cacc_s[...].astype(jnp.bfloat16), (((1,), (0,)), ((), ())),
            preferred_element_type=jnp.float32)
        cnt_s[...] = jnp.sum(cntm, axis=1, keepdims=True)
        macc[...] = jnp.zeros_like(macc)
        tacc[...] = jnp.zeros_like(tacc)

    @pl.when(i > NBLK)
    def _():
        simn2 = jax.lax.dot_general(
            bfm2_ref[...], q_ref[...], (((1,), (1,)), ((), ())),
            preferred_element_type=jnp.float32)
        t = simn2 + jnp.float32(2.0 + 1e-6)
        mae = (t * jax.lax.rsqrt(t)).astype(jnp.bfloat16)
        lab = labels_ref[0, 0, :]
        p16 = p_s[...].astype(jnp.int16)
        mask = p16 == lab[None, :]
        macc[...] += _lanegroup_sum(
            jnp.where(mask, mae, jnp.bfloat16(0.0))).astype(jnp.float32)
        tacc[...] += _lanegroup_sum(mae).astype(jnp.float32)

        @pl.when(i == 2 * NBLK)
        def _():
            m = jnp.sum(macc[...], axis=1)
            t_ = jnp.sum(tacc[...], axis=1)
            cnt = cnt_s[:, 0]
            minent = jnp.mean(m / (cnt + 1e-6))
            inter = jnp.mean((t_ - m) / ((jnp.float32(K) - cnt) + 1e-6))
            out_ref[...] = jnp.broadcast_to(minent + 2.0 - inter, (1, 1))


def _qblk(i):
    return jnp.where(i < NBLK, i, jnp.maximum(i - NBLK - 1, 0))


def kernel(batch_feature, queue_emb_copy, info_label):
    labels3 = info_label.astype(jnp.int16).reshape(K // KBLK, 1, KBLK)
    q16 = queue_emb_copy.astype(jnp.bfloat16)
    bfm2 = (batch_feature * -2.0).astype(jnp.bfloat16)

    out = pl.pallas_call(
        _fused_kernel,
        grid=(2 * NBLK + 1,),
        in_specs=[
            pl.BlockSpec((1, 1, KBLK), lambda i: (_qblk(i), 0, 0)),
            pl.BlockSpec((KBLK, D), lambda i: (_qblk(i), 0)),
            pl.BlockSpec((B, D), lambda i: (0, 0)),
        ],
        out_specs=pl.BlockSpec((1, 1), lambda i: (0, 0)),
        out_shape=jax.ShapeDtypeStruct((1, 1), jnp.float32),
        scratch_shapes=[
            pltpu.VMEM((CPAD, D), jnp.float32),
            pltpu.VMEM((CPAD, 128), jnp.float32),
            pltpu.VMEM((B, 1), jnp.int32),
            pltpu.VMEM((B, 1), jnp.float32),
            pltpu.VMEM((B, 128), jnp.float32),
            pltpu.VMEM((B, 128), jnp.float32),
        ],
    )(labels3, q16, bfm2)

    return out[0, 0]
